# Initial kernel scaffold; baseline (speedup 1.0000x reference)
import jax
import jax.numpy as jnp
from jax.experimental import pallas as pl


def kernel(feature_indices, feature_offsets, feature_weights, edge_index, emb_table, W1, b1, Wmu, bmu, Wls, bls):
    raise NotImplementedError("write your pallas kernel here")


# speedup vs baseline: 21.1216x; 21.1216x over previous
import functools

import jax
import jax.numpy as jnp
from jax import lax
from jax.experimental import pallas as pl
from jax.experimental.pallas import tpu as pltpu
from jax.experimental.pallas import tpu_sc as plsc

N = 50000
E = 800000
D = 64
O = 32

NC = 2
NS = 16
NW = NC * NS

N_PAD = 50176
E_PAD = 802816
ROWS_PER_TILE = N_PAD // NW
ROWS_PER_SUB = N_PAD // NS
EROWS = E_PAD // 128
DEG_BLKS = EROWS // NW // 14
PROP_BLKS = EROWS // NS // 14

_mesh = plsc.VectorSubcoreMesh(
    core_axis_name="c", subcore_axis_name="s", num_cores=NC, num_subcores=NS)
_sc_params = pltpu.CompilerParams(use_tc_tiling_on_sc=False)


@functools.partial(
    pl.kernel,
    out_type=[
        jax.ShapeDtypeStruct((N_PAD, D), jnp.float32),
        jax.ShapeDtypeStruct((NC * N_PAD,), jnp.float32),
    ],
    mesh=_mesh,
    scratch_types=[
        pltpu.VMEM((14, 112), jnp.int32),
        pltpu.VMEM((112, D), jnp.float32),
        pltpu.VMEM((14, 128), jnp.int32),
        pltpu.VMEM((128,), jnp.float32),
        pltpu.VMEM((112,), jnp.float32),
        pltpu.VMEM_SHARED((N_PAD,), jnp.float32),
        pltpu.SemaphoreType.DMA,
    ],
    compiler_params=_sc_params,
)
def _sc_gather_deg(table, fi2, dst2, rows_out, deg_out,
                   gidx, growbuf, didx, ones_v, zbuf, accd, sem):
    cid = lax.axis_index("c")
    sid = lax.axis_index("s")
    wid = sid * NC + cid

    pltpu.sync_copy(fi2.at[pl.ds(wid * 14, 14)], gidx)
    row_base = wid * ROWS_PER_TILE

    @pl.loop(0, 14)
    def _(j):
        pltpu.async_copy(table.at[gidx.at[j]], growbuf, sem).wait()
        pltpu.sync_copy(growbuf, rows_out.at[pl.ds(row_base + j * 112, 112)])

    for t in range(7):
        zbuf[pl.ds(t * 16, 16)] = jnp.zeros((16,), jnp.float32)
    for t in range(8):
        ones_v[pl.ds(t * 16, 16)] = jnp.full((16,), 1.0, jnp.float32)
    zb = sid * ROWS_PER_SUB

    @pl.loop(0, 28)
    def _(i):
        pltpu.sync_copy(zbuf, accd.at[pl.ds(zb + i * 112, 112)])

    plsc.subcore_barrier()
    eb = wid * (14 * DEG_BLKS)

    @pl.loop(0, DEG_BLKS)
    def _(b):
        pltpu.sync_copy(dst2.at[pl.ds(eb + b * 14, 14)], didx)
        for j in range(14):
            pltpu.sync_copy(ones_v, accd.at[didx.at[j]], add=True)

    plsc.subcore_barrier()
    pltpu.sync_copy(accd.at[pl.ds(zb, ROWS_PER_SUB)],
                    deg_out.at[pl.ds(cid * N_PAD + zb, ROWS_PER_SUB)])


@functools.partial(
    pl.kernel,
    out_type=jax.ShapeDtypeStruct((NC * N_PAD, O), jnp.float32),
    mesh=_mesh,
    scratch_types=[
        pltpu.VMEM((14, 128), jnp.int32),
        pltpu.VMEM((14, 128), jnp.int32),
        pltpu.VMEM((128, O), jnp.float32),
        pltpu.VMEM_SHARED((N_PAD, O), jnp.float32),
        pltpu.SemaphoreType.DMA,
    ],
    compiler_params=_sc_params,
)
def _sc_propagate(xs, srcf, dst2, y_out, sidx, didx, rowbuf, acc, sem):
    cid = lax.axis_index("c")
    sid = lax.axis_index("s")
    rb = sid * ROWS_PER_SUB

    pltpu.sync_copy(xs.at[pl.ds(cid * N_PAD + rb, ROWS_PER_SUB)],
                    acc.at[pl.ds(rb, ROWS_PER_SUB)])
    plsc.subcore_barrier()

    @pl.loop(0, PROP_BLKS)
    def _(b):
        base = sid * (14 * PROP_BLKS) + b * 14
        pltpu.sync_copy(srcf.at[pl.ds(cid * EROWS + base, 14)], sidx)
        pltpu.sync_copy(dst2.at[pl.ds(base, 14)], didx)
        for j in range(14):
            pltpu.async_copy(xs.at[sidx.at[j]], rowbuf, sem).wait()
            pltpu.sync_copy(rowbuf, acc.at[didx.at[j]], add=True)

    plsc.subcore_barrier()
    pltpu.sync_copy(acc.at[pl.ds(rb, ROWS_PER_SUB)],
                    y_out.at[pl.ds(cid * N_PAD + rb, ROWS_PER_SUB)])


R = 3136
GRID = N_PAD // R


def _tc1_body(rows_ref, fw_ref, deg_ref, w1_ref, xs_ref, dinv_ref):
    x = rows_ref[...] * fw_ref[...]
    nrm = jnp.maximum(jnp.sqrt(jnp.sum(x * x, axis=1, keepdims=True)), 1e-12)
    xn = x / nrm
    dinv = lax.rsqrt(1.0 + deg_ref[0] + deg_ref[1])
    h = jnp.dot(xn, w1_ref[...], preferred_element_type=jnp.float32) * dinv
    xs_ref[0] = h[:, :O]
    xs_ref[1] = h[:, O:]
    dinv_ref[...] = dinv


def _tc1(rows, fw2, deg3, w1):
    return pl.pallas_call(
        _tc1_body,
        grid=(GRID,),
        in_specs=[
            pl.BlockSpec((R, D), lambda i: (i, 0)),
            pl.BlockSpec((R, 1), lambda i: (i, 0)),
            pl.BlockSpec((NC, R, 1), lambda i: (0, i, 0)),
            pl.BlockSpec((D, D), lambda i: (0, 0)),
        ],
        out_specs=[
            pl.BlockSpec((NC, R, O), lambda i: (0, i, 0)),
            pl.BlockSpec((R, 1), lambda i: (i, 0)),
        ],
        out_shape=[
            jax.ShapeDtypeStruct((NC, N_PAD, O), jnp.float32),
            jax.ShapeDtypeStruct((N_PAD, 1), jnp.float32),
        ],
    )(rows, fw2, deg3, w1)


def _tc2_body(y_ref, dinv_ref, b1_ref, wc_ref, xs_ref):
    dinv = dinv_ref[...]
    p = jnp.concatenate([y_ref[0], y_ref[1]], axis=1)
    x1 = jnp.maximum(p * dinv + b1_ref[...], 0.0)
    h = jnp.dot(x1, wc_ref[...], preferred_element_type=jnp.float32) * dinv
    xs_ref[0] = h[:, :O]
    xs_ref[1] = h[:, O:]


def _tc2(y3, dinv2, b1_2d, wcat):
    return pl.pallas_call(
        _tc2_body,
        grid=(GRID,),
        in_specs=[
            pl.BlockSpec((NC, R, O), lambda i: (0, i, 0)),
            pl.BlockSpec((R, 1), lambda i: (i, 0)),
            pl.BlockSpec((1, D), lambda i: (0, 0)),
            pl.BlockSpec((D, D), lambda i: (0, 0)),
        ],
        out_specs=pl.BlockSpec((NC, R, O), lambda i: (0, i, 0)),
        out_shape=jax.ShapeDtypeStruct((NC, N_PAD, O), jnp.float32),
    )(y3, dinv2, b1_2d, wcat)


def _tc3_body(y_ref, dinv_ref, bmu_ref, bls_ref, nz_ref, z_ref):
    dinv = dinv_ref[...]
    mu = y_ref[0] * dinv + bmu_ref[...]
    ls = y_ref[1] * dinv + bls_ref[...]
    z_ref[...] = mu + nz_ref[...] * jnp.exp(ls)


def _tc3(y3, dinv2, bmu_2d, bls_2d, nz):
    return pl.pallas_call(
        _tc3_body,
        grid=(GRID,),
        in_specs=[
            pl.BlockSpec((NC, R, O), lambda i: (0, i, 0)),
            pl.BlockSpec((R, 1), lambda i: (i, 0)),
            pl.BlockSpec((1, O), lambda i: (0, 0)),
            pl.BlockSpec((1, O), lambda i: (0, 0)),
            pl.BlockSpec((R, O), lambda i: (i, 0)),
        ],
        out_specs=pl.BlockSpec((R, O), lambda i: (i, 0)),
        out_shape=jax.ShapeDtypeStruct((N_PAD, O), jnp.float32),
    )(y3, dinv2, bmu_2d, bls_2d, nz)


def kernel(feature_indices, feature_offsets, feature_weights, edge_index,
           emb_table, W1, b1, Wmu, bmu, Wls, bls):
    del feature_offsets
    fi2 = jnp.pad(feature_indices, (0, N_PAD - N)).reshape(N_PAD // 112, 112)
    fw2 = jnp.pad(feature_weights, (0, N_PAD - N)).reshape(N_PAD, 1)
    src = jnp.pad(edge_index[0], (0, E_PAD - E), constant_values=N)
    dst = jnp.pad(edge_index[1], (0, E_PAD - E), constant_values=N)
    srcf = jnp.concatenate([src, src + N_PAD]).reshape(NC * EROWS, 128)
    dst2 = dst.reshape(EROWS, 128)

    rows, deg = _sc_gather_deg(emb_table, fi2, dst2)
    deg3 = deg.reshape(NC, N_PAD, 1)
    xs1, dinv2 = _tc1(rows, fw2, deg3, W1)

    y1 = _sc_propagate(xs1.reshape(NC * N_PAD, O), srcf, dst2)
    wcat = jnp.concatenate([Wmu, Wls], axis=1)
    xs2 = _tc2(y1.reshape(NC, N_PAD, O), dinv2, b1.reshape(1, D), wcat)

    y2 = _sc_propagate(xs2.reshape(NC * N_PAD, O), srcf, dst2)
    noise = jax.random.normal(jax.random.key(42), (N, O), dtype=jnp.float32)
    nz = jnp.pad(noise, ((0, N_PAD - N), (0, 0)))
    z = _tc3(y2.reshape(NC, N_PAD, O), dinv2, bmu.reshape(1, O),
             bls.reshape(1, O), nz)
    return z[:N]


# speedup vs baseline: 36.2828x; 1.7178x over previous
import functools

import jax
import jax.numpy as jnp
from jax import lax
from jax.experimental import pallas as pl
from jax.experimental.pallas import tpu as pltpu
from jax.experimental.pallas import tpu_sc as plsc

N = 50000
E = 800000
D = 64
O = 32

NC = 2
NS = 16
NW = NC * NS

N_PAD = 50176
E_PAD = 802816
ROWS_PER_TILE = N_PAD // NW
ROWS_PER_SUB = N_PAD // NS
EROWS = E_PAD // 128
DEG_BLKS = EROWS // NW // 14
PROP_BLKS = EROWS // NS // 14

_mesh = plsc.VectorSubcoreMesh(
    core_axis_name="c", subcore_axis_name="s", num_cores=NC, num_subcores=NS)
_sc_params = pltpu.CompilerParams(use_tc_tiling_on_sc=False)


@functools.partial(
    pl.kernel,
    out_type=[
        jax.ShapeDtypeStruct((N_PAD, D), jnp.float32),
        jax.ShapeDtypeStruct((NC * N_PAD,), jnp.float32),
    ],
    mesh=_mesh,
    scratch_types=[
        pltpu.VMEM((14, 112), jnp.int32),
        pltpu.VMEM((112, D), jnp.float32),
        pltpu.VMEM((14, 128), jnp.int32),
        pltpu.VMEM((128,), jnp.float32),
        pltpu.VMEM((112,), jnp.float32),
        pltpu.VMEM_SHARED((N_PAD,), jnp.float32),
        pltpu.SemaphoreType.DMA,
    ],
    compiler_params=_sc_params,
)
def _sc_gather_deg(table, fi2, dst2, rows_out, deg_out,
                   gidx, growbuf, didx, ones_v, zbuf, accd, sem):
    cid = lax.axis_index("c")
    sid = lax.axis_index("s")
    wid = sid * NC + cid

    pltpu.sync_copy(fi2.at[pl.ds(wid * 14, 14)], gidx)
    row_base = wid * ROWS_PER_TILE

    @pl.loop(0, 14)
    def _(j):
        pltpu.async_copy(table.at[gidx.at[j]], growbuf, sem).wait()
        pltpu.sync_copy(growbuf, rows_out.at[pl.ds(row_base + j * 112, 112)])

    for t in range(7):
        zbuf[pl.ds(t * 16, 16)] = jnp.zeros((16,), jnp.float32)
    for t in range(8):
        ones_v[pl.ds(t * 16, 16)] = jnp.full((16,), 1.0, jnp.float32)
    zb = sid * ROWS_PER_SUB

    @pl.loop(0, 28)
    def _(i):
        pltpu.sync_copy(zbuf, accd.at[pl.ds(zb + i * 112, 112)])

    plsc.subcore_barrier()
    eb = wid * (14 * DEG_BLKS)

    @pl.loop(0, DEG_BLKS)
    def _(b):
        pltpu.sync_copy(dst2.at[pl.ds(eb + b * 14, 14)], didx)
        for j in range(14):
            pltpu.sync_copy(ones_v, accd.at[didx.at[j]], add=True)

    plsc.subcore_barrier()
    pltpu.sync_copy(accd.at[pl.ds(zb, ROWS_PER_SUB)],
                    deg_out.at[pl.ds(cid * N_PAD + zb, ROWS_PER_SUB)])


@functools.partial(
    pl.kernel,
    out_type=jax.ShapeDtypeStruct((NC * N_PAD, O), jnp.float32),
    mesh=_mesh,
    scratch_types=[
        pltpu.VMEM((2, 14, 128), jnp.int32),
        pltpu.VMEM((2, 14, 128), jnp.int32),
        pltpu.VMEM((5, 128, O), jnp.float32),
        pltpu.VMEM_SHARED((N_PAD, O), jnp.float32),
        pltpu.SemaphoreType.DMA((5,)),
        pltpu.SemaphoreType.DMA((2,)),
        pltpu.SemaphoreType.DMA,
    ],
    compiler_params=_sc_params,
)
def _sc_propagate(xs, srcf, dst2, y_out, sidx, didx, bufs, acc,
                  gsem, isem, sem0):
    cid = lax.axis_index("c")
    sid = lax.axis_index("s")
    rb = sid * ROWS_PER_SUB
    ebase = sid * (14 * PROP_BLKS)
    NCHUNK = 14 * PROP_BLKS
    RING = 5

    def fire_idx(b, h):
        base = ebase + b * 14
        pltpu.async_copy(srcf.at[pl.ds(cid * EROWS + base, 14)],
                         sidx.at[h], isem.at[h])
        pltpu.async_copy(dst2.at[pl.ds(base, 14)], didx.at[h], isem.at[h])

    def wait_idx(h):
        pltpu.make_async_copy(srcf.at[pl.ds(0, 14)], sidx.at[h],
                              isem.at[h]).wait()
        pltpu.make_async_copy(dst2.at[pl.ds(0, 14)], didx.at[h],
                              isem.at[h]).wait()

    def fire_gather(c, s):
        b = lax.div(c, 14)
        j = lax.rem(c, 14)
        h = lax.rem(b, 2)
        pltpu.async_copy(xs.at[sidx.at[h].at[j]], bufs.at[s], gsem.at[s])

    def wait_gather(s):
        pltpu.make_async_copy(xs.at[pl.ds(0, 128)], bufs.at[s],
                              gsem.at[s]).wait()

    fire_idx(0, 0)
    fire_idx(1, 1)
    pltpu.async_copy(xs.at[pl.ds(cid * N_PAD + rb, ROWS_PER_SUB)],
                     acc.at[pl.ds(rb, ROWS_PER_SUB)], sem0)
    wait_idx(0)
    for s in range(RING):
        fire_gather(s, s)
    pltpu.make_async_copy(xs.at[pl.ds(0, ROWS_PER_SUB)],
                          acc.at[pl.ds(rb, ROWS_PER_SUB)], sem0).wait()
    plsc.subcore_barrier()

    @pl.loop(0, NCHUNK)
    def _(c):
        b = lax.div(c, 14)
        j = lax.rem(c, 14)
        h = lax.rem(b, 2)
        s = lax.rem(c, RING)
        wait_gather(s)
        pltpu.sync_copy(bufs.at[s], acc.at[didx.at[h].at[j]], add=True)

        @pl.when(c + RING < NCHUNK)
        def _():
            fire_gather(c + RING, s)

        @pl.when(jnp.logical_and(j == 8, b + 1 < PROP_BLKS))
        def _():
            wait_idx(lax.rem(b + 1, 2))

        @pl.when(jnp.logical_and(j == 13, b + 2 < PROP_BLKS))
        def _():
            fire_idx(b + 2, h)

    plsc.subcore_barrier()
    pltpu.sync_copy(acc.at[pl.ds(rb, ROWS_PER_SUB)],
                    y_out.at[pl.ds(cid * N_PAD + rb, ROWS_PER_SUB)])


R = 3136
GRID = N_PAD // R


def _tc1_body(rows_ref, fw_ref, deg_ref, w1_ref, xs_ref, dinv_ref):
    x = rows_ref[...] * fw_ref[...]
    nrm = jnp.maximum(jnp.sqrt(jnp.sum(x * x, axis=1, keepdims=True)), 1e-12)
    xn = x / nrm
    dinv = lax.rsqrt(1.0 + deg_ref[0] + deg_ref[1])
    h = jnp.dot(xn, w1_ref[...], preferred_element_type=jnp.float32) * dinv
    xs_ref[0] = h[:, :O]
    xs_ref[1] = h[:, O:]
    dinv_ref[...] = dinv


def _tc1(rows, fw2, deg3, w1):
    return pl.pallas_call(
        _tc1_body,
        grid=(GRID,),
        in_specs=[
            pl.BlockSpec((R, D), lambda i: (i, 0)),
            pl.BlockSpec((R, 1), lambda i: (i, 0)),
            pl.BlockSpec((NC, R, 1), lambda i: (0, i, 0)),
            pl.BlockSpec((D, D), lambda i: (0, 0)),
        ],
        out_specs=[
            pl.BlockSpec((NC, R, O), lambda i: (0, i, 0)),
            pl.BlockSpec((R, 1), lambda i: (i, 0)),
        ],
        out_shape=[
            jax.ShapeDtypeStruct((NC, N_PAD, O), jnp.float32),
            jax.ShapeDtypeStruct((N_PAD, 1), jnp.float32),
        ],
    )(rows, fw2, deg3, w1)


def _tc2_body(y_ref, dinv_ref, b1_ref, wc_ref, xs_ref):
    dinv = dinv_ref[...]
    p = jnp.concatenate([y_ref[0], y_ref[1]], axis=1)
    x1 = jnp.maximum(p * dinv + b1_ref[...], 0.0)
    h = jnp.dot(x1, wc_ref[...], preferred_element_type=jnp.float32) * dinv
    xs_ref[0] = h[:, :O]
    xs_ref[1] = h[:, O:]


def _tc2(y3, dinv2, b1_2d, wcat):
    return pl.pallas_call(
        _tc2_body,
        grid=(GRID,),
        in_specs=[
            pl.BlockSpec((NC, R, O), lambda i: (0, i, 0)),
            pl.BlockSpec((R, 1), lambda i: (i, 0)),
            pl.BlockSpec((1, D), lambda i: (0, 0)),
            pl.BlockSpec((D, D), lambda i: (0, 0)),
        ],
        out_specs=pl.BlockSpec((NC, R, O), lambda i: (0, i, 0)),
        out_shape=jax.ShapeDtypeStruct((NC, N_PAD, O), jnp.float32),
    )(y3, dinv2, b1_2d, wcat)


def _tc3_body(y_ref, dinv_ref, bmu_ref, bls_ref, nz_ref, z_ref):
    dinv = dinv_ref[...]
    mu = y_ref[0] * dinv + bmu_ref[...]
    ls = y_ref[1] * dinv + bls_ref[...]
    z_ref[...] = mu + nz_ref[...] * jnp.exp(ls)


def _tc3(y3, dinv2, bmu_2d, bls_2d, nz):
    return pl.pallas_call(
        _tc3_body,
        grid=(GRID,),
        in_specs=[
            pl.BlockSpec((NC, R, O), lambda i: (0, i, 0)),
            pl.BlockSpec((R, 1), lambda i: (i, 0)),
            pl.BlockSpec((1, O), lambda i: (0, 0)),
            pl.BlockSpec((1, O), lambda i: (0, 0)),
            pl.BlockSpec((R, O), lambda i: (i, 0)),
        ],
        out_specs=pl.BlockSpec((R, O), lambda i: (i, 0)),
        out_shape=jax.ShapeDtypeStruct((N_PAD, O), jnp.float32),
    )(y3, dinv2, bmu_2d, bls_2d, nz)


def kernel(feature_indices, feature_offsets, feature_weights, edge_index,
           emb_table, W1, b1, Wmu, bmu, Wls, bls):
    del feature_offsets
    fi2 = jnp.pad(feature_indices, (0, N_PAD - N)).reshape(N_PAD // 112, 112)
    fw2 = jnp.pad(feature_weights, (0, N_PAD - N)).reshape(N_PAD, 1)
    src = jnp.pad(edge_index[0], (0, E_PAD - E), constant_values=N)
    dst = jnp.pad(edge_index[1], (0, E_PAD - E), constant_values=N)
    srcf = jnp.concatenate([src, src + N_PAD]).reshape(NC * EROWS, 128)
    dst2 = dst.reshape(EROWS, 128)

    rows, deg = _sc_gather_deg(emb_table, fi2, dst2)
    deg3 = deg.reshape(NC, N_PAD, 1)
    xs1, dinv2 = _tc1(rows, fw2, deg3, W1)

    y1 = _sc_propagate(xs1.reshape(NC * N_PAD, O), srcf, dst2)
    wcat = jnp.concatenate([Wmu, Wls], axis=1)
    xs2 = _tc2(y1.reshape(NC, N_PAD, O), dinv2, b1.reshape(1, D), wcat)

    y2 = _sc_propagate(xs2.reshape(NC * N_PAD, O), srcf, dst2)
    noise = jax.random.normal(jax.random.key(42), (N, O), dtype=jnp.float32)
    nz = jnp.pad(noise, ((0, N_PAD - N), (0, 0)))
    z = _tc3(y2.reshape(NC, N_PAD, O), dinv2, bmu.reshape(1, O),
             bls.reshape(1, O), nz)
    return z[:N]


# speedup vs baseline: 37.2692x; 1.0272x over previous
import functools

import jax
import jax.numpy as jnp
from jax import lax
from jax.experimental import pallas as pl
from jax.experimental.pallas import tpu as pltpu
from jax.experimental.pallas import tpu_sc as plsc

N = 50000
E = 800000
D = 64
O = 32

NC = 2
NS = 16
NW = NC * NS

N_PAD = 50176
E_PAD = 802816
ROWS_PER_TILE = N_PAD // NW
ROWS_PER_SUB = N_PAD // NS
EROWS = E_PAD // 128
DEG_BLKS = EROWS // NW // 14
PROP_BLKS = EROWS // NS // 14

_mesh = plsc.VectorSubcoreMesh(
    core_axis_name="c", subcore_axis_name="s", num_cores=NC, num_subcores=NS)
_sc_params = pltpu.CompilerParams(use_tc_tiling_on_sc=False)


@functools.partial(
    pl.kernel,
    out_type=[
        jax.ShapeDtypeStruct((N_PAD, D), jnp.float32),
        jax.ShapeDtypeStruct((NC * N_PAD,), jnp.float32),
    ],
    mesh=_mesh,
    scratch_types=[
        pltpu.VMEM((14, 112), jnp.int32),
        pltpu.VMEM((4, 112, D), jnp.float32),
        pltpu.VMEM((2, 14, 128), jnp.int32),
        pltpu.VMEM((128,), jnp.float32),
        pltpu.VMEM((112,), jnp.float32),
        pltpu.VMEM_SHARED((N_PAD,), jnp.float32),
        pltpu.SemaphoreType.DMA((4,)),
        pltpu.SemaphoreType.DMA((2,)),
    ],
    compiler_params=_sc_params,
)
def _sc_gather_deg(table, fi2, dst2, rows_out, deg_out,
                   gidx, gbufs, didx, ones_v, zbuf, accd, gsem, isem):
    cid = lax.axis_index("c")
    sid = lax.axis_index("s")
    wid = sid * NC + cid
    row_base = wid * ROWS_PER_TILE
    eb = wid * (14 * DEG_BLKS)

    def fire_dst(b, h):
        pltpu.async_copy(dst2.at[pl.ds(eb + b * 14, 14)], didx.at[h],
                         isem.at[h])

    def wait_dst(h):
        pltpu.make_async_copy(dst2.at[pl.ds(0, 14)], didx.at[h],
                              isem.at[h]).wait()

    def fire_gather(j, s):
        pltpu.async_copy(table.at[gidx.at[j]], gbufs.at[s], gsem.at[s])

    def wait_gather(s):
        pltpu.make_async_copy(table.at[pl.ds(0, 112)], gbufs.at[s],
                              gsem.at[s]).wait()

    pltpu.sync_copy(fi2.at[pl.ds(wid * 14, 14)], gidx)
    for s in range(4):
        fire_gather(s, s)
    fire_dst(0, 0)
    fire_dst(1, 1)

    for t in range(7):
        zbuf[pl.ds(t * 16, 16)] = jnp.zeros((16,), jnp.float32)
    for t in range(8):
        ones_v[pl.ds(t * 16, 16)] = jnp.full((16,), 1.0, jnp.float32)
    zb = sid * ROWS_PER_SUB

    @pl.loop(0, 28)
    def _(i):
        pltpu.sync_copy(zbuf, accd.at[pl.ds(zb + i * 112, 112)])

    @pl.loop(0, 14)
    def _(j):
        s = lax.rem(j, 4)
        wait_gather(s)
        pltpu.sync_copy(gbufs.at[s], rows_out.at[pl.ds(row_base + j * 112,
                                                       112)])

        @pl.when(j + 4 < 14)
        def _():
            fire_gather(j + 4, s)

    plsc.subcore_barrier()

    @pl.loop(0, DEG_BLKS)
    def _(b):
        h = lax.rem(b, 2)
        wait_dst(h)
        for j in range(14):
            pltpu.sync_copy(ones_v, accd.at[didx.at[h].at[j]], add=True)

        @pl.when(b + 2 < DEG_BLKS)
        def _():
            fire_dst(b + 2, h)

    plsc.subcore_barrier()
    pltpu.sync_copy(accd.at[pl.ds(zb, ROWS_PER_SUB)],
                    deg_out.at[pl.ds(cid * N_PAD + zb, ROWS_PER_SUB)])


@functools.partial(
    pl.kernel,
    out_type=jax.ShapeDtypeStruct((NC * N_PAD, O), jnp.float32),
    mesh=_mesh,
    scratch_types=[
        pltpu.VMEM((2, 14, 128), jnp.int32),
        pltpu.VMEM((2, 14, 128), jnp.int32),
        pltpu.VMEM((5, 128, O), jnp.float32),
        pltpu.VMEM_SHARED((N_PAD, O), jnp.float32),
        pltpu.SemaphoreType.DMA((5,)),
        pltpu.SemaphoreType.DMA((2,)),
        pltpu.SemaphoreType.DMA,
    ],
    compiler_params=_sc_params,
)
def _sc_propagate(xs, srcf, dst2, y_out, sidx, didx, bufs, acc,
                  gsem, isem, sem0):
    cid = lax.axis_index("c")
    sid = lax.axis_index("s")
    rb = sid * ROWS_PER_SUB
    ebase = sid * (14 * PROP_BLKS)
    NCHUNK = 14 * PROP_BLKS
    RING = 5

    xs_half = xs.at[pl.ds(cid * N_PAD, N_PAD)]

    def fire_idx(b, h):
        base = ebase + b * 14
        pltpu.async_copy(srcf.at[pl.ds(base, 14)], sidx.at[h], isem.at[h])
        pltpu.async_copy(dst2.at[pl.ds(base, 14)], didx.at[h], isem.at[h])

    def wait_idx(h):
        pltpu.make_async_copy(srcf.at[pl.ds(0, 14)], sidx.at[h],
                              isem.at[h]).wait()
        pltpu.make_async_copy(dst2.at[pl.ds(0, 14)], didx.at[h],
                              isem.at[h]).wait()

    def fire_gather(c, s):
        b = lax.div(c, 14)
        j = lax.rem(c, 14)
        h = lax.rem(b, 2)
        pltpu.async_copy(xs_half.at[sidx.at[h].at[j]], bufs.at[s],
                         gsem.at[s])

    def wait_gather(s):
        pltpu.make_async_copy(xs.at[pl.ds(0, 128)], bufs.at[s],
                              gsem.at[s]).wait()

    fire_idx(0, 0)
    fire_idx(1, 1)
    pltpu.async_copy(xs.at[pl.ds(cid * N_PAD + rb, ROWS_PER_SUB)],
                     acc.at[pl.ds(rb, ROWS_PER_SUB)], sem0)
    wait_idx(0)
    for s in range(RING):
        fire_gather(s, s)
    pltpu.make_async_copy(xs.at[pl.ds(0, ROWS_PER_SUB)],
                          acc.at[pl.ds(rb, ROWS_PER_SUB)], sem0).wait()
    plsc.subcore_barrier()

    @pl.loop(0, NCHUNK)
    def _(c):
        b = lax.div(c, 14)
        j = lax.rem(c, 14)
        h = lax.rem(b, 2)
        s = lax.rem(c, RING)
        wait_gather(s)
        pltpu.sync_copy(bufs.at[s], acc.at[didx.at[h].at[j]], add=True)

        @pl.when(c + RING < NCHUNK)
        def _():
            fire_gather(c + RING, s)

        @pl.when(jnp.logical_and(j == 8, b + 1 < PROP_BLKS))
        def _():
            wait_idx(lax.rem(b + 1, 2))

        @pl.when(jnp.logical_and(j == 13, b + 2 < PROP_BLKS))
        def _():
            fire_idx(b + 2, h)

    plsc.subcore_barrier()
    pltpu.sync_copy(acc.at[pl.ds(rb, ROWS_PER_SUB)],
                    y_out.at[pl.ds(cid * N_PAD + rb, ROWS_PER_SUB)])


R = 3136
GRID = N_PAD // R


def _tc1_body(rows_ref, fw_ref, deg_ref, w1_ref, xs_ref, dinv_ref):
    x = rows_ref[...] * fw_ref[...]
    nrm = jnp.maximum(jnp.sqrt(jnp.sum(x * x, axis=1, keepdims=True)), 1e-12)
    xn = x / nrm
    dinv = lax.rsqrt(1.0 + deg_ref[0] + deg_ref[1])
    h = jnp.dot(xn, w1_ref[...], preferred_element_type=jnp.float32) * dinv
    xs_ref[0] = h[:, :O]
    xs_ref[1] = h[:, O:]
    dinv_ref[...] = dinv


def _tc1(rows, fw2, deg3, w1):
    return pl.pallas_call(
        _tc1_body,
        grid=(GRID,),
        in_specs=[
            pl.BlockSpec((R, D), lambda i: (i, 0)),
            pl.BlockSpec((R, 1), lambda i: (i, 0)),
            pl.BlockSpec((NC, R, 1), lambda i: (0, i, 0)),
            pl.BlockSpec((D, D), lambda i: (0, 0)),
        ],
        out_specs=[
            pl.BlockSpec((NC, R, O), lambda i: (0, i, 0)),
            pl.BlockSpec((R, 1), lambda i: (i, 0)),
        ],
        out_shape=[
            jax.ShapeDtypeStruct((NC, N_PAD, O), jnp.float32),
            jax.ShapeDtypeStruct((N_PAD, 1), jnp.float32),
        ],
    )(rows, fw2, deg3, w1)


def _tc2_body(y_ref, dinv_ref, b1_ref, wc_ref, xs_ref):
    dinv = dinv_ref[...]
    p = jnp.concatenate([y_ref[0], y_ref[1]], axis=1)
    x1 = jnp.maximum(p * dinv + b1_ref[...], 0.0)
    h = jnp.dot(x1, wc_ref[...], preferred_element_type=jnp.float32) * dinv
    xs_ref[0] = h[:, :O]
    xs_ref[1] = h[:, O:]


def _tc2(y3, dinv2, b1_2d, wcat):
    return pl.pallas_call(
        _tc2_body,
        grid=(GRID,),
        in_specs=[
            pl.BlockSpec((NC, R, O), lambda i: (0, i, 0)),
            pl.BlockSpec((R, 1), lambda i: (i, 0)),
            pl.BlockSpec((1, D), lambda i: (0, 0)),
            pl.BlockSpec((D, D), lambda i: (0, 0)),
        ],
        out_specs=pl.BlockSpec((NC, R, O), lambda i: (0, i, 0)),
        out_shape=jax.ShapeDtypeStruct((NC, N_PAD, O), jnp.float32),
    )(y3, dinv2, b1_2d, wcat)


def _tc3_body(y_ref, dinv_ref, bmu_ref, bls_ref, nz_ref, z_ref):
    dinv = dinv_ref[...]
    mu = y_ref[0] * dinv + bmu_ref[...]
    ls = y_ref[1] * dinv + bls_ref[...]
    z_ref[...] = mu + nz_ref[...] * jnp.exp(ls)


def _tc3(y3, dinv2, bmu_2d, bls_2d, nz):
    return pl.pallas_call(
        _tc3_body,
        grid=(GRID,),
        in_specs=[
            pl.BlockSpec((NC, R, O), lambda i: (0, i, 0)),
            pl.BlockSpec((R, 1), lambda i: (i, 0)),
            pl.BlockSpec((1, O), lambda i: (0, 0)),
            pl.BlockSpec((1, O), lambda i: (0, 0)),
            pl.BlockSpec((R, O), lambda i: (i, 0)),
        ],
        out_specs=pl.BlockSpec((R, O), lambda i: (i, 0)),
        out_shape=jax.ShapeDtypeStruct((N_PAD, O), jnp.float32),
    )(y3, dinv2, bmu_2d, bls_2d, nz)


def kernel(feature_indices, feature_offsets, feature_weights, edge_index,
           emb_table, W1, b1, Wmu, bmu, Wls, bls):
    del feature_offsets
    fi2 = jnp.pad(feature_indices, (0, N_PAD - N)).reshape(N_PAD // 112, 112)
    fw2 = jnp.pad(feature_weights, (0, N_PAD - N)).reshape(N_PAD, 1)
    src = jnp.pad(edge_index[0], (0, E_PAD - E), constant_values=N)
    dst = jnp.pad(edge_index[1], (0, E_PAD - E), constant_values=N)
    srcf = src.reshape(EROWS, 128)
    dst2 = dst.reshape(EROWS, 128)

    rows, deg = _sc_gather_deg(emb_table, fi2, dst2)
    deg3 = deg.reshape(NC, N_PAD, 1)
    xs1, dinv2 = _tc1(rows, fw2, deg3, W1)

    y1 = _sc_propagate(xs1.reshape(NC * N_PAD, O), srcf, dst2)
    wcat = jnp.concatenate([Wmu, Wls], axis=1)
    xs2 = _tc2(y1.reshape(NC, N_PAD, O), dinv2, b1.reshape(1, D), wcat)

    y2 = _sc_propagate(xs2.reshape(NC * N_PAD, O), srcf, dst2)
    noise = jax.random.normal(jax.random.key(42), (N, O), dtype=jnp.float32)
    nz = jnp.pad(noise, ((0, N_PAD - N), (0, 0)))
    z = _tc3(y2.reshape(NC, N_PAD, O), dinv2, bmu.reshape(1, O),
             bls.reshape(1, O), nz)
    return z[:N]


# speedup vs baseline: 38.2712x; 1.0269x over previous
import functools

import jax
import jax.numpy as jnp
from jax import lax
from jax.experimental import pallas as pl
from jax.experimental.pallas import tpu as pltpu
from jax.experimental.pallas import tpu_sc as plsc

N = 50000
E = 800000
D = 64
O = 32

NC = 2
NS = 16
NW = NC * NS

N_PAD = 50176
E_PAD = 802816
ROWS_PER_TILE = N_PAD // NW
ROWS_PER_SUB = N_PAD // NS
EROWS = E_PAD // 128
DEG_BLKS = EROWS // NW // 14
PROP_BLKS = EROWS // NS // 14

_mesh = plsc.VectorSubcoreMesh(
    core_axis_name="c", subcore_axis_name="s", num_cores=NC, num_subcores=NS)
_sc_params = pltpu.CompilerParams(use_tc_tiling_on_sc=False)


@functools.partial(
    pl.kernel,
    out_type=[
        jax.ShapeDtypeStruct((N_PAD, D), jnp.float32),
        jax.ShapeDtypeStruct((NC * N_PAD,), jnp.float32),
    ],
    mesh=_mesh,
    scratch_types=[
        pltpu.VMEM((14, 112), jnp.int32),
        pltpu.VMEM((4, 112, D), jnp.float32),
        pltpu.VMEM((2, 14, 128), jnp.int32),
        pltpu.VMEM((128,), jnp.float32),
        pltpu.VMEM((112,), jnp.float32),
        pltpu.VMEM_SHARED((N_PAD,), jnp.float32),
        pltpu.SemaphoreType.DMA((4,)),
        pltpu.SemaphoreType.DMA((2,)),
    ],
    compiler_params=_sc_params,
)
def _sc_gather_deg(table, fi2, edges, rows_out, deg_out,
                   gidx, gbufs, didx, ones_v, zbuf, accd, gsem, isem):
    dst2 = edges.at[1]
    cid = lax.axis_index("c")
    sid = lax.axis_index("s")
    wid = sid * NC + cid
    row_base = wid * ROWS_PER_TILE
    eb = wid * (14 * DEG_BLKS)

    def fire_dst(b, h):
        pltpu.async_copy(dst2.at[pl.ds(eb + b * 14, 14)], didx.at[h],
                         isem.at[h])

    def wait_dst(h):
        pltpu.make_async_copy(dst2.at[pl.ds(0, 14)], didx.at[h],
                              isem.at[h]).wait()

    def fire_gather(j, s):
        pltpu.async_copy(table.at[gidx.at[j]], gbufs.at[s], gsem.at[s])

    def wait_gather(s):
        pltpu.make_async_copy(table.at[pl.ds(0, 112)], gbufs.at[s],
                              gsem.at[s]).wait()

    pltpu.sync_copy(fi2.at[pl.ds(wid * 14, 14)], gidx)
    for s in range(4):
        fire_gather(s, s)
    fire_dst(0, 0)
    fire_dst(1, 1)

    for t in range(7):
        zbuf[pl.ds(t * 16, 16)] = jnp.zeros((16,), jnp.float32)
    for t in range(8):
        ones_v[pl.ds(t * 16, 16)] = jnp.full((16,), 1.0, jnp.float32)
    zb = sid * ROWS_PER_SUB

    @pl.loop(0, 28)
    def _(i):
        pltpu.sync_copy(zbuf, accd.at[pl.ds(zb + i * 112, 112)])

    @pl.loop(0, 14)
    def _(j):
        s = lax.rem(j, 4)
        wait_gather(s)
        pltpu.sync_copy(gbufs.at[s], rows_out.at[pl.ds(row_base + j * 112,
                                                       112)])

        @pl.when(j + 4 < 14)
        def _():
            fire_gather(j + 4, s)

    plsc.subcore_barrier()

    @pl.loop(0, DEG_BLKS)
    def _(b):
        h = lax.rem(b, 2)
        wait_dst(h)
        for j in range(14):
            pltpu.sync_copy(ones_v, accd.at[didx.at[h].at[j]], add=True)

        @pl.when(b + 2 < DEG_BLKS)
        def _():
            fire_dst(b + 2, h)

    plsc.subcore_barrier()
    pltpu.sync_copy(accd.at[pl.ds(zb, ROWS_PER_SUB)],
                    deg_out.at[pl.ds(cid * N_PAD + zb, ROWS_PER_SUB)])


@functools.partial(
    pl.kernel,
    out_type=jax.ShapeDtypeStruct((NC, N_PAD, O), jnp.float32),
    mesh=_mesh,
    scratch_types=[
        pltpu.VMEM((2, 14, 128), jnp.int32),
        pltpu.VMEM((2, 14, 128), jnp.int32),
        pltpu.VMEM((5, 128, O), jnp.float32),
        pltpu.VMEM_SHARED((N_PAD, O), jnp.float32),
        pltpu.SemaphoreType.DMA((5,)),
        pltpu.SemaphoreType.DMA((2,)),
        pltpu.SemaphoreType.DMA,
    ],
    compiler_params=_sc_params,
)
def _sc_propagate(xs, edges, y_out, sidx, didx, bufs, acc,
                  gsem, isem, sem0):
    cid = lax.axis_index("c")
    sid = lax.axis_index("s")
    rb = sid * ROWS_PER_SUB
    ebase = sid * (14 * PROP_BLKS)
    NCHUNK = 14 * PROP_BLKS
    RING = 5

    srcf = edges.at[0]
    dst2 = edges.at[1]
    xs_half = xs.at[cid]

    def fire_idx(b, h):
        base = ebase + b * 14
        pltpu.async_copy(srcf.at[pl.ds(base, 14)], sidx.at[h], isem.at[h])
        pltpu.async_copy(dst2.at[pl.ds(base, 14)], didx.at[h], isem.at[h])

    def wait_idx(h):
        pltpu.make_async_copy(srcf.at[pl.ds(0, 14)], sidx.at[h],
                              isem.at[h]).wait()
        pltpu.make_async_copy(dst2.at[pl.ds(0, 14)], didx.at[h],
                              isem.at[h]).wait()

    def fire_gather(c, s):
        b = lax.div(c, 14)
        j = lax.rem(c, 14)
        h = lax.rem(b, 2)
        pltpu.async_copy(xs_half.at[sidx.at[h].at[j]], bufs.at[s],
                         gsem.at[s])

    def wait_gather(s):
        pltpu.make_async_copy(xs_half.at[pl.ds(0, 128)], bufs.at[s],
                              gsem.at[s]).wait()

    fire_idx(0, 0)
    fire_idx(1, 1)
    pltpu.async_copy(xs_half.at[pl.ds(rb, ROWS_PER_SUB)],
                     acc.at[pl.ds(rb, ROWS_PER_SUB)], sem0)
    wait_idx(0)
    for s in range(RING):
        fire_gather(s, s)
    pltpu.make_async_copy(xs_half.at[pl.ds(0, ROWS_PER_SUB)],
                          acc.at[pl.ds(rb, ROWS_PER_SUB)], sem0).wait()
    plsc.subcore_barrier()

    @pl.loop(0, NCHUNK)
    def _(c):
        b = lax.div(c, 14)
        j = lax.rem(c, 14)
        h = lax.rem(b, 2)
        s = lax.rem(c, RING)
        wait_gather(s)
        pltpu.sync_copy(bufs.at[s], acc.at[didx.at[h].at[j]], add=True)

        @pl.when(c + RING < NCHUNK)
        def _():
            fire_gather(c + RING, s)

        @pl.when(jnp.logical_and(j == 8, b + 1 < PROP_BLKS))
        def _():
            wait_idx(lax.rem(b + 1, 2))

        @pl.when(jnp.logical_and(j == 13, b + 2 < PROP_BLKS))
        def _():
            fire_idx(b + 2, h)

    plsc.subcore_barrier()
    pltpu.sync_copy(acc.at[pl.ds(rb, ROWS_PER_SUB)],
                    y_out.at[cid].at[pl.ds(rb, ROWS_PER_SUB)])


R = 3136
GRID = N_PAD // R


def _tc1_body(rows_ref, fw_ref, deg_ref, w1_ref, xs_ref, dinv_ref):
    x = rows_ref[...] * fw_ref[...]
    nrm = jnp.maximum(jnp.sqrt(jnp.sum(x * x, axis=1, keepdims=True)), 1e-12)
    xn = x / nrm
    dinv = lax.rsqrt(1.0 + deg_ref[0] + deg_ref[1])
    h = jnp.dot(xn, w1_ref[...], preferred_element_type=jnp.float32) * dinv
    xs_ref[0] = h[:, :O]
    xs_ref[1] = h[:, O:]
    dinv_ref[...] = dinv


def _tc1(rows, fw2, deg3, w1):
    return pl.pallas_call(
        _tc1_body,
        grid=(GRID,),
        in_specs=[
            pl.BlockSpec((R, D), lambda i: (i, 0)),
            pl.BlockSpec((R, 1), lambda i: (i, 0)),
            pl.BlockSpec((NC, R, 1), lambda i: (0, i, 0)),
            pl.BlockSpec((D, D), lambda i: (0, 0)),
        ],
        out_specs=[
            pl.BlockSpec((NC, R, O), lambda i: (0, i, 0)),
            pl.BlockSpec((R, 1), lambda i: (i, 0)),
        ],
        out_shape=[
            jax.ShapeDtypeStruct((NC, N_PAD, O), jnp.float32),
            jax.ShapeDtypeStruct((N_PAD, 1), jnp.float32),
        ],
    )(rows, fw2, deg3, w1)


def _tc2_body(y_ref, dinv_ref, b1_ref, wc_ref, xs_ref):
    dinv = dinv_ref[...]
    p = jnp.concatenate([y_ref[0], y_ref[1]], axis=1)
    x1 = jnp.maximum(p * dinv + b1_ref[...], 0.0)
    h = jnp.dot(x1, wc_ref[...], preferred_element_type=jnp.float32) * dinv
    xs_ref[0] = h[:, :O]
    xs_ref[1] = h[:, O:]


def _tc2(y3, dinv2, b1_2d, wcat):
    return pl.pallas_call(
        _tc2_body,
        grid=(GRID,),
        in_specs=[
            pl.BlockSpec((NC, R, O), lambda i: (0, i, 0)),
            pl.BlockSpec((R, 1), lambda i: (i, 0)),
            pl.BlockSpec((1, D), lambda i: (0, 0)),
            pl.BlockSpec((D, D), lambda i: (0, 0)),
        ],
        out_specs=pl.BlockSpec((NC, R, O), lambda i: (0, i, 0)),
        out_shape=jax.ShapeDtypeStruct((NC, N_PAD, O), jnp.float32),
    )(y3, dinv2, b1_2d, wcat)


def _tc3_body(y_ref, dinv_ref, bmu_ref, bls_ref, nz_ref, z_ref):
    dinv = dinv_ref[...]
    mu = y_ref[0] * dinv + bmu_ref[...]
    ls = y_ref[1] * dinv + bls_ref[...]
    z_ref[...] = mu + nz_ref[...] * jnp.exp(ls)


R3 = 2000
GRID3 = N // R3


def _tc3(y3, dinv2, bmu_2d, bls_2d, nz):
    return pl.pallas_call(
        _tc3_body,
        grid=(GRID3,),
        in_specs=[
            pl.BlockSpec((NC, R3, O), lambda i: (0, i, 0)),
            pl.BlockSpec((R3, 1), lambda i: (i, 0)),
            pl.BlockSpec((1, O), lambda i: (0, 0)),
            pl.BlockSpec((1, O), lambda i: (0, 0)),
            pl.BlockSpec((R3, O), lambda i: (i, 0)),
        ],
        out_specs=pl.BlockSpec((R3, O), lambda i: (i, 0)),
        out_shape=jax.ShapeDtypeStruct((N, O), jnp.float32),
    )(y3, dinv2, bmu_2d, bls_2d, nz)


def kernel(feature_indices, feature_offsets, feature_weights, edge_index,
           emb_table, W1, b1, Wmu, bmu, Wls, bls):
    del feature_offsets
    fi2 = jnp.pad(feature_indices, (0, N_PAD - N)).reshape(N_PAD // 112, 112)
    fw2 = jnp.pad(feature_weights, (0, N_PAD - N)).reshape(N_PAD, 1)
    edges = jnp.pad(edge_index, ((0, 0), (0, E_PAD - E)),
                    constant_values=N).reshape(2, EROWS, 128)

    rows, deg = _sc_gather_deg(emb_table, fi2, edges)
    deg3 = deg.reshape(NC, N_PAD, 1)
    xs1, dinv2 = _tc1(rows, fw2, deg3, W1)

    y1 = _sc_propagate(xs1, edges)
    wcat = jnp.concatenate([Wmu, Wls], axis=1)
    xs2 = _tc2(y1, dinv2, b1.reshape(1, D), wcat)

    y2 = _sc_propagate(xs2, edges)
    noise = jax.random.normal(jax.random.key(42), (N, O), dtype=jnp.float32)
    return _tc3(y2, dinv2, bmu.reshape(1, O), bls.reshape(1, O), noise)
